# baseline (speedup 1.0000x reference)
import jax
import jax.numpy as jnp
from jax.experimental import pallas as pl

_INP = 128
_OUT = 128
_DEG = 32
_N = 10000
_BN = 200


def _body(mb_ref, nf_ref, w1_ref, w2_ref, b_ref, out_ref):
    m = jnp.min(mb_ref[...], axis=1)
    acc = jnp.dot(m, w1_ref[...], preferred_element_type=jnp.float32)
    acc = acc + jnp.dot(nf_ref[...], w2_ref[...], preferred_element_type=jnp.float32)
    out_ref[...] = acc + b_ref[...]


def kernel(mailbox_h, node_feat, W, b):
    W1T = W[:, :_INP].T
    W2T = W[:, _INP:].T
    b2 = b.reshape(1, _OUT)
    return pl.pallas_call(
        _body,
        grid=(_N // _BN,),
        in_specs=[
            pl.BlockSpec((_BN, _DEG, _INP), lambda i: (i, 0, 0)),
            pl.BlockSpec((_BN, _INP), lambda i: (i, 0)),
            pl.BlockSpec((_INP, _OUT), lambda i: (0, 0)),
            pl.BlockSpec((_INP, _OUT), lambda i: (0, 0)),
            pl.BlockSpec((1, _OUT), lambda i: (0, 0)),
        ],
        out_specs=pl.BlockSpec((_BN, _OUT), lambda i: (i, 0)),
        out_shape=jax.ShapeDtypeStruct((_N, _OUT), jnp.float32),
    )(mailbox_h, node_feat, W1T, W2T, b2)


# speedup vs baseline: 1.2229x; 1.2229x over previous
import jax
import jax.numpy as jnp
from jax.experimental import pallas as pl

_INP = 128
_OUT = 128
_DEG = 32
_N = 10000
_BN = 400


def _body(mb_ref, nf_ref, w1_ref, w2_ref, b_ref, out_ref):
    m = jnp.min(mb_ref[...], axis=1)
    acc = jnp.dot(m, w1_ref[...], preferred_element_type=jnp.float32)
    acc = acc + jnp.dot(nf_ref[...], w2_ref[...], preferred_element_type=jnp.float32)
    out_ref[...] = acc + b_ref[...]


def kernel(mailbox_h, node_feat, W, b):
    W1T = W[:, :_INP].T
    W2T = W[:, _INP:].T
    b2 = b.reshape(1, _OUT)
    return pl.pallas_call(
        _body,
        grid=(_N // _BN,),
        in_specs=[
            pl.BlockSpec((_BN, _DEG, _INP), lambda i: (i, 0, 0)),
            pl.BlockSpec((_BN, _INP), lambda i: (i, 0)),
            pl.BlockSpec((_INP, _OUT), lambda i: (0, 0)),
            pl.BlockSpec((_INP, _OUT), lambda i: (0, 0)),
            pl.BlockSpec((1, _OUT), lambda i: (0, 0)),
        ],
        out_specs=pl.BlockSpec((_BN, _OUT), lambda i: (i, 0)),
        out_shape=jax.ShapeDtypeStruct((_N, _OUT), jnp.float32),
    )(mailbox_h, node_feat, W1T, W2T, b2)
